# Initial kernel scaffold; baseline (speedup 1.0000x reference)
import jax
import jax.numpy as jnp
from jax.experimental import pallas as pl
from jax.experimental.pallas import tpu as pltpu

N_PAD = 128
TE = 4096
G = 2


def _round_up(x, m):
    return ((x + m - 1) // m) * m


def _iota_sub_bf16():
    return jax.lax.broadcasted_iota(
        jnp.float32, (N_PAD, 1), 0).astype(jnp.bfloat16)


def _scatter_kernel(row_ref, col_ref, val_ref, acc_ref):
    k = pl.program_id(1)

    @pl.when(k == 0)
    def _():
        acc_ref[...] = jnp.zeros_like(acc_ref)

    sub = _iota_sub_bf16()
    row_b = row_ref[...].astype(jnp.bfloat16)
    col_b = col_ref[...].astype(jnp.bfloat16)
    val_b = val_ref[...].astype(jnp.bfloat16)
    rv = jnp.where(row_b == sub, val_b, jnp.bfloat16(0))
    c = (col_b == sub).astype(jnp.bfloat16)
    acc_ref[0] += jax.lax.dot_general(
        rv, c, (((1,), (1,)), ((), ())),
        preferred_element_type=jnp.float32)


def _gather_kernel(parts_ref, row_ref, col_ref, out_ref, s_ref):
    k = pl.program_id(1)

    @pl.when(k == 0)
    def _():
        a = parts_ref[...].sum(axis=0)
        s_ref[...] = ((a + a.T) * 0.5).astype(jnp.bfloat16)

    sub = _iota_sub_bf16()
    col_b = col_ref[...].astype(jnp.bfloat16)
    c = (col_b == sub).astype(jnp.bfloat16)
    g = jnp.dot(s_ref[...], c,
                preferred_element_type=jnp.float32)
    row_b = row_ref[...].astype(jnp.bfloat16)
    out_ref[...] = jnp.where(row_b == sub, g, 0.0).sum(
        axis=0, keepdims=True)


def _sym_edge_mask(row, col, val):
    E = row.shape[0]
    chunk = G * TE
    E_pad = _round_up(max(E, 1), chunk)
    pad = E_pad - E
    num_tiles = E_pad // TE
    k_g = num_tiles // G
    k_c = num_tiles // 2

    row_p = jnp.pad(row.astype(jnp.int32), (0, pad)).reshape(1, E_pad)
    col_p = jnp.pad(col.astype(jnp.int32), (0, pad)).reshape(1, E_pad)
    val_p = jnp.pad(val.astype(jnp.float32), (0, pad)).reshape(1, E_pad)

    parts = pl.pallas_call(
        _scatter_kernel,
        out_shape=jax.ShapeDtypeStruct((G, N_PAD, N_PAD), jnp.float32),
        grid=(G, k_g),
        in_specs=[
            pl.BlockSpec((1, TE), lambda g, k: (0, g * k_g + k)),
            pl.BlockSpec((1, TE), lambda g, k: (0, g * k_g + k)),
            pl.BlockSpec((1, TE), lambda g, k: (0, g * k_g + k)),
        ],
        out_specs=pl.BlockSpec((1, N_PAD, N_PAD), lambda g, k: (g, 0, 0)),
        compiler_params=pltpu.CompilerParams(
            dimension_semantics=("parallel", "arbitrary")),
    )(row_p, col_p, val_p)

    out = pl.pallas_call(
        _gather_kernel,
        out_shape=jax.ShapeDtypeStruct((1, E_pad), jnp.float32),
        grid=(2, k_c),
        in_specs=[
            pl.BlockSpec((G, N_PAD, N_PAD), lambda c, k: (0, 0, 0)),
            pl.BlockSpec((1, TE), lambda c, k: (0, c * k_c + k)),
            pl.BlockSpec((1, TE), lambda c, k: (0, c * k_c + k)),
        ],
        out_specs=pl.BlockSpec((1, TE), lambda c, k: (0, c * k_c + k)),
        scratch_shapes=[pltpu.VMEM((N_PAD, N_PAD), jnp.bfloat16)],
        compiler_params=pltpu.CompilerParams(
            dimension_semantics=("parallel", "arbitrary")),
    )(parts, row_p, col_p)

    return out[0, :E]


def kernel(x, edge_index, edge_mask, node_feat_mask):
    del x, node_feat_mask
    row = edge_index[0].astype(jnp.int32)
    col = edge_index[1].astype(jnp.int32)
    return _sym_edge_mask(row, col, edge_mask.astype(jnp.float32))


# speedup vs baseline: 1.3324x; 1.3324x over previous
import jax
import jax.numpy as jnp
from jax.experimental import pallas as pl
from jax.experimental.pallas import tpu as pltpu

N_PAD = 128
TE = 4096
G = 2


def _round_up(x, m):
    return ((x + m - 1) // m) * m


def _iota_sub_bf16():
    return jax.lax.broadcasted_iota(
        jnp.int32, (N_PAD, 1), 0).astype(jnp.bfloat16)


def _scatter_kernel(row_ref, col_ref, val_ref, acc_ref):
    k = pl.program_id(1)

    @pl.when(k == 0)
    def _():
        acc_ref[...] = jnp.zeros_like(acc_ref)

    sub = _iota_sub_bf16()
    row_b = row_ref[...].astype(jnp.bfloat16)
    col_b = col_ref[...].astype(jnp.bfloat16)
    val_b = val_ref[...].astype(jnp.bfloat16)
    rv = jnp.where(row_b == sub, val_b, jnp.bfloat16(0))
    c = (col_b == sub).astype(jnp.bfloat16)
    acc_ref[0] += jax.lax.dot_general(
        rv, c, (((1,), (1,)), ((), ())),
        preferred_element_type=jnp.float32)


def _gather_kernel(parts_ref, row_ref, col_ref, out_ref, s_ref):
    k = pl.program_id(1)

    @pl.when(k == 0)
    def _():
        a = parts_ref[...].sum(axis=0)
        s_ref[...] = ((a + a.T) * 0.5).astype(jnp.bfloat16)

    sub = _iota_sub_bf16()
    col_b = col_ref[...].astype(jnp.bfloat16)
    c = (col_b == sub).astype(jnp.bfloat16)
    g = jnp.dot(s_ref[...], c,
                preferred_element_type=jnp.float32)
    row_b = row_ref[...].astype(jnp.bfloat16)
    out_ref[...] = jnp.where(row_b == sub, g, 0.0).sum(
        axis=0, keepdims=True)


def _sym_edge_mask(row, col, val):
    E = row.shape[0]
    chunk = G * TE
    E_pad = _round_up(max(E, 1), chunk)
    pad = E_pad - E
    num_tiles = E_pad // TE
    k_g = num_tiles // G
    k_c = num_tiles // 2

    row_p = jnp.pad(row.astype(jnp.int32), (0, pad)).reshape(1, E_pad)
    col_p = jnp.pad(col.astype(jnp.int32), (0, pad)).reshape(1, E_pad)
    val_p = jnp.pad(val.astype(jnp.float32), (0, pad)).reshape(1, E_pad)

    parts = pl.pallas_call(
        _scatter_kernel,
        out_shape=jax.ShapeDtypeStruct((G, N_PAD, N_PAD), jnp.float32),
        grid=(G, k_g),
        in_specs=[
            pl.BlockSpec((1, TE), lambda g, k: (0, g * k_g + k)),
            pl.BlockSpec((1, TE), lambda g, k: (0, g * k_g + k)),
            pl.BlockSpec((1, TE), lambda g, k: (0, g * k_g + k)),
        ],
        out_specs=pl.BlockSpec((1, N_PAD, N_PAD), lambda g, k: (g, 0, 0)),
        compiler_params=pltpu.CompilerParams(
            dimension_semantics=("parallel", "arbitrary")),
    )(row_p, col_p, val_p)

    out = pl.pallas_call(
        _gather_kernel,
        out_shape=jax.ShapeDtypeStruct((1, E_pad), jnp.float32),
        grid=(2, k_c),
        in_specs=[
            pl.BlockSpec((G, N_PAD, N_PAD), lambda c, k: (0, 0, 0)),
            pl.BlockSpec((1, TE), lambda c, k: (0, c * k_c + k)),
            pl.BlockSpec((1, TE), lambda c, k: (0, c * k_c + k)),
        ],
        out_specs=pl.BlockSpec((1, TE), lambda c, k: (0, c * k_c + k)),
        scratch_shapes=[pltpu.VMEM((N_PAD, N_PAD), jnp.bfloat16)],
        compiler_params=pltpu.CompilerParams(
            dimension_semantics=("parallel", "arbitrary")),
    )(parts, row_p, col_p)

    return out[0, :E]


def kernel(x, edge_index, edge_mask, node_feat_mask):
    del x, node_feat_mask
    row = edge_index[0].astype(jnp.int32)
    col = edge_index[1].astype(jnp.int32)
    return _sym_edge_mask(row, col, edge_mask.astype(jnp.float32))


# speedup vs baseline: 1.7499x; 1.3134x over previous
import jax
import jax.numpy as jnp
from jax.experimental import pallas as pl
from jax.experimental.pallas import tpu as pltpu

N_PAD = 128
TE = 4096
CH = 1024


def _round_up(x, m):
    return ((x + m - 1) // m) * m


def _scatter_kernel(row_ref, col_ref, val_ref, acc_ref):
    k = pl.program_id(0)

    @pl.when(k == 0)
    def _():
        acc_ref[...] = jnp.zeros_like(acc_ref)

    sub = jax.lax.broadcasted_iota(jnp.int32, (N_PAD, 1), 0)
    rv = jnp.where(row_ref[...] == sub, val_ref[...], 0.0).astype(jnp.bfloat16)
    c = jnp.where(col_ref[...] == sub, 1.0, 0.0).astype(jnp.bfloat16)
    acc_ref[...] += jax.lax.dot_general(
        rv, c, (((1,), (1,)), ((), ())),
        preferred_element_type=jnp.float32)


def _gather_kernel(a_ref, row_ref, col_ref, out_ref, s_ref):
    k = pl.program_id(0)

    @pl.when(k == 0)
    def _():
        a = a_ref[...]
        s_ref[...] = ((a + a.T) * 0.5).astype(jnp.bfloat16)

    sub = jax.lax.broadcasted_iota(jnp.int32, (N_PAD, 1), 0)
    c = jnp.where(col_ref[...] == sub, 1.0, 0.0).astype(jnp.bfloat16)
    g = jnp.dot(s_ref[...], c,
                preferred_element_type=jnp.float32)
    out_ref[...] = jnp.where(row_ref[...] == sub, g, 0.0).sum(
        axis=0, keepdims=True)


def _sym_edge_mask(row, col, val):
    E = row.shape[0]
    E_pad = _round_up(max(E, 1), TE)
    pad = E_pad - E
    num_tiles = E_pad // TE

    row_p = jnp.pad(row.astype(jnp.int32), (0, pad)).reshape(1, E_pad)
    col_p = jnp.pad(col.astype(jnp.int32), (0, pad)).reshape(1, E_pad)
    val_p = jnp.pad(val.astype(jnp.float32), (0, pad)).reshape(1, E_pad)

    a = pl.pallas_call(
        _scatter_kernel,
        out_shape=jax.ShapeDtypeStruct((N_PAD, N_PAD), jnp.float32),
        grid=(num_tiles,),
        in_specs=[
            pl.BlockSpec((1, TE), lambda k: (0, k)),
            pl.BlockSpec((1, TE), lambda k: (0, k)),
            pl.BlockSpec((1, TE), lambda k: (0, k)),
        ],
        out_specs=pl.BlockSpec((N_PAD, N_PAD), lambda k: (0, 0)),
        compiler_params=pltpu.CompilerParams(
            dimension_semantics=("arbitrary",)),
    )(row_p, col_p, val_p)

    out = pl.pallas_call(
        _gather_kernel,
        out_shape=jax.ShapeDtypeStruct((1, E_pad), jnp.float32),
        grid=(num_tiles,),
        in_specs=[
            pl.BlockSpec((N_PAD, N_PAD), lambda k: (0, 0)),
            pl.BlockSpec((1, TE), lambda k: (0, k)),
            pl.BlockSpec((1, TE), lambda k: (0, k)),
        ],
        out_specs=pl.BlockSpec((1, TE), lambda k: (0, k)),
        scratch_shapes=[pltpu.VMEM((N_PAD, N_PAD), jnp.bfloat16)],
        compiler_params=pltpu.CompilerParams(
            dimension_semantics=("arbitrary",)),
    )(a, row_p, col_p)

    return out[0, :E]


def kernel(x, edge_index, edge_mask, node_feat_mask):
    del x, node_feat_mask
    row = edge_index[0].astype(jnp.int32)
    col = edge_index[1].astype(jnp.int32)
    return _sym_edge_mask(row, col, edge_mask.astype(jnp.float32))


# speedup vs baseline: 2.7246x; 1.5570x over previous
import jax
import jax.numpy as jnp
from jax.experimental import pallas as pl
from jax.experimental.pallas import tpu as pltpu

N_PAD = 128
TE = 8192
CH = 1024


def _round_up(x, m):
    return ((x + m - 1) // m) * m


def _scatter_kernel(row_ref, col_ref, val_ref, acc_ref):
    k = pl.program_id(0)

    @pl.when(k == 0)
    def _():
        acc_ref[...] = jnp.zeros_like(acc_ref)

    sub = jax.lax.broadcasted_iota(jnp.int32, (N_PAD, 1), 0)
    rv = jnp.where(row_ref[...] == sub, val_ref[...], 0.0).astype(jnp.bfloat16)
    c = jnp.where(col_ref[...] == sub, 1.0, 0.0).astype(jnp.bfloat16)
    acc_ref[...] += jax.lax.dot_general(
        rv, c, (((1,), (1,)), ((), ())),
        preferred_element_type=jnp.float32)


def _gather_kernel(a_ref, row_ref, col_ref, out_ref, s_ref):
    k = pl.program_id(0)

    @pl.when(k == 0)
    def _():
        a = a_ref[...]
        s_ref[...] = ((a + a.T) * 0.5).astype(jnp.bfloat16)

    sub = jax.lax.broadcasted_iota(jnp.int32, (N_PAD, 1), 0)
    c = jnp.where(col_ref[...] == sub, 1.0, 0.0).astype(jnp.bfloat16)
    g = jnp.dot(s_ref[...], c,
                preferred_element_type=jnp.float32)
    out_ref[...] = jnp.where(row_ref[...] == sub, g, 0.0).sum(
        axis=0, keepdims=True)


def _sym_edge_mask(row, col, val):
    E = row.shape[0]
    E_pad = _round_up(max(E, 1), TE)
    pad = E_pad - E
    num_tiles = E_pad // TE

    row_p = jnp.pad(row.astype(jnp.int32), (0, pad)).reshape(1, E_pad)
    col_p = jnp.pad(col.astype(jnp.int32), (0, pad)).reshape(1, E_pad)
    val_p = jnp.pad(val.astype(jnp.float32), (0, pad)).reshape(1, E_pad)

    a = pl.pallas_call(
        _scatter_kernel,
        out_shape=jax.ShapeDtypeStruct((N_PAD, N_PAD), jnp.float32),
        grid=(num_tiles,),
        in_specs=[
            pl.BlockSpec((1, TE), lambda k: (0, k)),
            pl.BlockSpec((1, TE), lambda k: (0, k)),
            pl.BlockSpec((1, TE), lambda k: (0, k)),
        ],
        out_specs=pl.BlockSpec((N_PAD, N_PAD), lambda k: (0, 0)),
        compiler_params=pltpu.CompilerParams(
            dimension_semantics=("arbitrary",)),
    )(row_p, col_p, val_p)

    out = pl.pallas_call(
        _gather_kernel,
        out_shape=jax.ShapeDtypeStruct((1, E_pad), jnp.float32),
        grid=(num_tiles,),
        in_specs=[
            pl.BlockSpec((N_PAD, N_PAD), lambda k: (0, 0)),
            pl.BlockSpec((1, TE), lambda k: (0, k)),
            pl.BlockSpec((1, TE), lambda k: (0, k)),
        ],
        out_specs=pl.BlockSpec((1, TE), lambda k: (0, k)),
        scratch_shapes=[pltpu.VMEM((N_PAD, N_PAD), jnp.bfloat16)],
        compiler_params=pltpu.CompilerParams(
            dimension_semantics=("arbitrary",)),
    )(a, row_p, col_p)

    return out[0, :E]


def kernel(x, edge_index, edge_mask, node_feat_mask):
    del x, node_feat_mask
    row = edge_index[0].astype(jnp.int32)
    col = edge_index[1].astype(jnp.int32)
    return _sym_edge_mask(row, col, edge_mask.astype(jnp.float32))


# speedup vs baseline: 3.6914x; 1.3549x over previous
import jax
import jax.numpy as jnp
from jax.experimental import pallas as pl
from jax.experimental.pallas import tpu as pltpu

N_PAD = 128
TE = 65536
CH = 1024


def _round_up(x, m):
    return ((x + m - 1) // m) * m


def _scatter_kernel(row_ref, col_ref, val_ref, acc_ref):
    k = pl.program_id(0)

    @pl.when(k == 0)
    def _():
        acc_ref[...] = jnp.zeros_like(acc_ref)

    sub = jax.lax.broadcasted_iota(jnp.int32, (N_PAD, 1), 0)
    rv = jnp.where(row_ref[...] == sub, val_ref[...], 0.0).astype(jnp.bfloat16)
    c = jnp.where(col_ref[...] == sub, 1.0, 0.0).astype(jnp.bfloat16)
    acc_ref[...] += jax.lax.dot_general(
        rv, c, (((1,), (1,)), ((), ())),
        preferred_element_type=jnp.float32)


def _gather_kernel(a_ref, row_ref, col_ref, out_ref, s_ref):
    k = pl.program_id(0)

    @pl.when(k == 0)
    def _():
        a = a_ref[...]
        s_ref[...] = ((a + a.T) * 0.5).astype(jnp.bfloat16)

    sub = jax.lax.broadcasted_iota(jnp.int32, (N_PAD, 1), 0)
    c = jnp.where(col_ref[...] == sub, 1.0, 0.0).astype(jnp.bfloat16)
    g = jnp.dot(s_ref[...], c,
                preferred_element_type=jnp.float32)
    out_ref[...] = jnp.where(row_ref[...] == sub, g, 0.0).sum(
        axis=0, keepdims=True)


def _sym_edge_mask(row, col, val):
    E = row.shape[0]
    E_pad = _round_up(max(E, 1), TE)
    pad = E_pad - E
    num_tiles = E_pad // TE

    row_p = jnp.pad(row.astype(jnp.int32), (0, pad)).reshape(1, E_pad)
    col_p = jnp.pad(col.astype(jnp.int32), (0, pad)).reshape(1, E_pad)
    val_p = jnp.pad(val.astype(jnp.float32), (0, pad)).reshape(1, E_pad)

    a = pl.pallas_call(
        _scatter_kernel,
        out_shape=jax.ShapeDtypeStruct((N_PAD, N_PAD), jnp.float32),
        grid=(num_tiles,),
        in_specs=[
            pl.BlockSpec((1, TE), lambda k: (0, k)),
            pl.BlockSpec((1, TE), lambda k: (0, k)),
            pl.BlockSpec((1, TE), lambda k: (0, k)),
        ],
        out_specs=pl.BlockSpec((N_PAD, N_PAD), lambda k: (0, 0)),
        compiler_params=pltpu.CompilerParams(
            dimension_semantics=("arbitrary",)),
    )(row_p, col_p, val_p)

    out = pl.pallas_call(
        _gather_kernel,
        out_shape=jax.ShapeDtypeStruct((1, E_pad), jnp.float32),
        grid=(num_tiles,),
        in_specs=[
            pl.BlockSpec((N_PAD, N_PAD), lambda k: (0, 0)),
            pl.BlockSpec((1, TE), lambda k: (0, k)),
            pl.BlockSpec((1, TE), lambda k: (0, k)),
        ],
        out_specs=pl.BlockSpec((1, TE), lambda k: (0, k)),
        scratch_shapes=[pltpu.VMEM((N_PAD, N_PAD), jnp.bfloat16)],
        compiler_params=pltpu.CompilerParams(
            dimension_semantics=("arbitrary",)),
    )(a, row_p, col_p)

    return out[0, :E]


def kernel(x, edge_index, edge_mask, node_feat_mask):
    del x, node_feat_mask
    row = edge_index[0].astype(jnp.int32)
    col = edge_index[1].astype(jnp.int32)
    return _sym_edge_mask(row, col, edge_mask.astype(jnp.float32))


# speedup vs baseline: 3.7553x; 1.0173x over previous
import jax
import jax.numpy as jnp
from jax.experimental import pallas as pl
from jax.experimental.pallas import tpu as pltpu

N_PAD = 128
TE = 131072
CH = 1024


def _round_up(x, m):
    return ((x + m - 1) // m) * m


def _scatter_kernel(row_ref, col_ref, val_ref, acc_ref):
    k = pl.program_id(0)

    @pl.when(k == 0)
    def _():
        acc_ref[...] = jnp.zeros_like(acc_ref)

    sub = jax.lax.broadcasted_iota(jnp.int32, (N_PAD, 1), 0)
    rv = jnp.where(row_ref[...] == sub, val_ref[...], 0.0).astype(jnp.bfloat16)
    c = jnp.where(col_ref[...] == sub, 1.0, 0.0).astype(jnp.bfloat16)
    acc_ref[...] += jax.lax.dot_general(
        rv, c, (((1,), (1,)), ((), ())),
        preferred_element_type=jnp.float32)


def _gather_kernel(a_ref, row_ref, col_ref, out_ref, s_ref):
    k = pl.program_id(0)

    @pl.when(k == 0)
    def _():
        a = a_ref[...]
        s_ref[...] = ((a + a.T) * 0.5).astype(jnp.bfloat16)

    sub = jax.lax.broadcasted_iota(jnp.int32, (N_PAD, 1), 0)
    c = jnp.where(col_ref[...] == sub, 1.0, 0.0).astype(jnp.bfloat16)
    g = jnp.dot(s_ref[...], c,
                preferred_element_type=jnp.float32)
    out_ref[...] = jnp.where(row_ref[...] == sub, g, 0.0).sum(
        axis=0, keepdims=True)


def _sym_edge_mask(row, col, val):
    E = row.shape[0]
    E_pad = _round_up(max(E, 1), TE)
    pad = E_pad - E
    num_tiles = E_pad // TE

    row_p = jnp.pad(row.astype(jnp.int32), (0, pad)).reshape(1, E_pad)
    col_p = jnp.pad(col.astype(jnp.int32), (0, pad)).reshape(1, E_pad)
    val_p = jnp.pad(val.astype(jnp.float32), (0, pad)).reshape(1, E_pad)

    a = pl.pallas_call(
        _scatter_kernel,
        out_shape=jax.ShapeDtypeStruct((N_PAD, N_PAD), jnp.float32),
        grid=(num_tiles,),
        in_specs=[
            pl.BlockSpec((1, TE), lambda k: (0, k)),
            pl.BlockSpec((1, TE), lambda k: (0, k)),
            pl.BlockSpec((1, TE), lambda k: (0, k)),
        ],
        out_specs=pl.BlockSpec((N_PAD, N_PAD), lambda k: (0, 0)),
        compiler_params=pltpu.CompilerParams(
            dimension_semantics=("arbitrary",)),
    )(row_p, col_p, val_p)

    out = pl.pallas_call(
        _gather_kernel,
        out_shape=jax.ShapeDtypeStruct((1, E_pad), jnp.float32),
        grid=(num_tiles,),
        in_specs=[
            pl.BlockSpec((N_PAD, N_PAD), lambda k: (0, 0)),
            pl.BlockSpec((1, TE), lambda k: (0, k)),
            pl.BlockSpec((1, TE), lambda k: (0, k)),
        ],
        out_specs=pl.BlockSpec((1, TE), lambda k: (0, k)),
        scratch_shapes=[pltpu.VMEM((N_PAD, N_PAD), jnp.bfloat16)],
        compiler_params=pltpu.CompilerParams(
            dimension_semantics=("arbitrary",)),
    )(a, row_p, col_p)

    return out[0, :E]


def kernel(x, edge_index, edge_mask, node_feat_mask):
    del x, node_feat_mask
    row = edge_index[0].astype(jnp.int32)
    col = edge_index[1].astype(jnp.int32)
    return _sym_edge_mask(row, col, edge_mask.astype(jnp.float32))


# speedup vs baseline: 3.8333x; 1.0208x over previous
import jax
import jax.numpy as jnp
from jax.experimental import pallas as pl
from jax.experimental.pallas import tpu as pltpu

N_PAD = 128
TE = 131072


def _round_up(x, m):
    return ((x + m - 1) // m) * m


def _scatter_kernel(row_ref, col_ref, val_ref, acc_ref):
    k = pl.program_id(0)

    @pl.when(k == 0)
    def _():
        acc_ref[...] = jnp.zeros_like(acc_ref)

    sub = jax.lax.broadcasted_iota(jnp.int32, (N_PAD, 1), 0)
    rv = jnp.where(row_ref[0] == sub, val_ref[...], 0.0).astype(jnp.bfloat16)
    c = jnp.where(col_ref[0] == sub, 1.0, 0.0).astype(jnp.bfloat16)
    acc_ref[...] += jax.lax.dot_general(
        rv, c, (((1,), (1,)), ((), ())),
        preferred_element_type=jnp.float32)


def _gather_kernel(a_ref, row_ref, col_ref, out_ref, s_ref):
    k = pl.program_id(0)

    @pl.when(k == 0)
    def _():
        a = a_ref[...]
        s_ref[...] = ((a + a.T) * 0.5).astype(jnp.bfloat16)

    sub = jax.lax.broadcasted_iota(jnp.int32, (N_PAD, 1), 0)
    c = jnp.where(col_ref[0] == sub, 1.0, 0.0).astype(jnp.bfloat16)
    g = jnp.dot(s_ref[...], c,
                preferred_element_type=jnp.float32)
    out_ref[...] = jnp.where(row_ref[0] == sub, g, 0.0).sum(
        axis=0, keepdims=True)


def _sym_edge_mask(edge_index, val):
    E = val.shape[0]
    E_pad = _round_up(max(E, 1), TE)
    pad = E_pad - E
    num_tiles = E_pad // TE

    ei = jnp.pad(edge_index.astype(jnp.int32), ((0, 0), (0, pad))).reshape(2, 1, E_pad)
    val_p = jnp.pad(val.astype(jnp.float32), (0, pad)).reshape(1, E_pad)

    a = pl.pallas_call(
        _scatter_kernel,
        out_shape=jax.ShapeDtypeStruct((N_PAD, N_PAD), jnp.float32),
        grid=(num_tiles,),
        in_specs=[
            pl.BlockSpec((1, 1, TE), lambda k: (0, 0, k)),
            pl.BlockSpec((1, 1, TE), lambda k: (1, 0, k)),
            pl.BlockSpec((1, TE), lambda k: (0, k)),
        ],
        out_specs=pl.BlockSpec((N_PAD, N_PAD), lambda k: (0, 0)),
        compiler_params=pltpu.CompilerParams(
            dimension_semantics=("arbitrary",)),
    )(ei, ei, val_p)

    out = pl.pallas_call(
        _gather_kernel,
        out_shape=jax.ShapeDtypeStruct((1, E_pad), jnp.float32),
        grid=(num_tiles,),
        in_specs=[
            pl.BlockSpec((N_PAD, N_PAD), lambda k: (0, 0)),
            pl.BlockSpec((1, 1, TE), lambda k: (0, 0, k)),
            pl.BlockSpec((1, 1, TE), lambda k: (1, 0, k)),
        ],
        out_specs=pl.BlockSpec((1, TE), lambda k: (0, k)),
        scratch_shapes=[pltpu.VMEM((N_PAD, N_PAD), jnp.bfloat16)],
        compiler_params=pltpu.CompilerParams(
            dimension_semantics=("arbitrary",)),
    )(a, ei, ei)

    return out[0, :E]


def kernel(x, edge_index, edge_mask, node_feat_mask):
    del x, node_feat_mask
    return _sym_edge_mask(edge_index, edge_mask)
